# baseline (speedup 1.0000x reference)
import functools

import jax
import jax.numpy as jnp
from jax import lax
from jax.experimental import pallas as pl
from jax.experimental.pallas import tpu as pltpu
from jax.experimental.pallas import tpu_sc as plsc

UNK_SIZE = 1000


def _lane_shuffle(x, perm_idx):
    return lax.gather(
        x, perm_idx[:, None],
        dimension_numbers=lax.GatherDimensionNumbers(
            offset_dims=(), collapsed_slice_dims=(0,), start_index_map=(0,)),
        slice_sizes=(1,),
        mode=lax.GatherScatterMode.PROMISE_IN_BOUNDS)


def kernel(context, glove_table, unk_table):
    B, T = context.shape
    V, D = glove_table.shape
    TOK = B * T

    info = plsc.get_sparse_core_info()
    NC, NS, L = info.num_cores, info.num_subcores, info.num_lanes
    NW = NC * NS
    per_w = TOK // NW
    S = 256
    n_chunks = per_w // S
    NB = 3
    K = 2
    n_main = (n_chunks // NB) * NB
    assert TOK == per_w * NW and per_w == n_chunks * S

    mesh = plsc.VectorSubcoreMesh(core_axis_name="c", subcore_axis_name="s")

    @functools.partial(
        pl.kernel,
        mesh=mesh,
        out_type=jax.ShapeDtypeStruct((TOK, D), jnp.float32),
        scratch_types=[
            pltpu.VMEM((per_w,), jnp.int32),
            pltpu.VMEM((NB, S, D), jnp.float32),
            pltpu.VMEM((n_chunks * L,), jnp.int32),
        ] + [pltpu.VMEM((S,), jnp.int32) for _ in range(NB)]
          + [pltpu.SemaphoreType.DMA] * (2 * NB),
    )
    def k(ctx_hbm, glove_hbm, unk_hbm, out_hbm,
          idx_v, rows_v, min_v, *rest):
        gidxs, sems = rest[:NB], rest[NB:]
        gsems, wsems = sems[:NB], sems[NB:]
        wid = lax.axis_index("s") * NC + lax.axis_index("c")
        w_base = wid * per_w
        iota = lax.iota(jnp.int32, L)

        pltpu.sync_copy(ctx_hbm.at[pl.ds(w_base, per_w)], idx_v)

        def pre_body(c, carry):
            acc = jnp.full((L,), jnp.int32(2**31 - 1), jnp.int32)
            for g in range(S // L):
                v = idx_v[pl.ds(c * S + g * L, L)]
                acc = jnp.minimum(acc, v)
            for sh in (1, 2, 4, 8):
                acc = jnp.minimum(acc, _lane_shuffle(acc, iota ^ sh))
            min_v[pl.ds(c * L, L)] = acc
            return carry

        lax.fori_loop(0, n_chunks, pre_body, 0)

        def start_gather(c, buf):
            for g in range(S // L):
                v = idx_v[pl.ds(c * S + g * L, L)]
                gidxs[buf][pl.ds(g * L, L)] = jnp.maximum(v - UNK_SIZE, 0)
            return pltpu.async_copy(
                glove_hbm.at[gidxs[buf]], rows_v.at[buf], gsems[buf])

        def patch(c, buf):
            mn = min_v[pl.ds(c * L, L)]

            @pl.when(mn[0] < UNK_SIZE)
            def _():
                def patch_group(g, carry):
                    vv = idx_v[pl.ds(c * S + g * L, L)]
                    for i in range(L):
                        uid = vv[i]

                        @pl.when(uid < UNK_SIZE)
                        def _(i=i, uid=uid):
                            pltpu.sync_copy(
                                unk_hbm.at[pl.ds(uid, 1)],
                                rows_v.at[buf].at[pl.ds(g * L + i, 1)],
                            )
                    return carry

                lax.fori_loop(0, S // L, patch_group, 0)

        for j in range(K):
            start_gather(j, j)

        def step_body(s, carry):
            for b in range(NB):
                c = s * NB + b
                b2 = (b + K) % NB

                @pl.when(c + K < n_chunks)
                def _(c=c, b2=b2):
                    @pl.when(c + K - NB >= 0)
                    def _():
                        pltpu.make_async_copy(
                            rows_v.at[b2],
                            out_hbm.at[pl.ds(w_base, S)],
                            wsems[b2]).wait()

                    start_gather(c + K, b2)

                pltpu.make_async_copy(
                    glove_hbm.at[gidxs[b]],
                    rows_v.at[b], gsems[b]).wait()
                patch(c, b)
                pltpu.async_copy(
                    rows_v.at[b],
                    out_hbm.at[pl.ds(w_base + c * S, S)], wsems[b])
            return carry

        lax.fori_loop(0, n_main // NB, step_body, 0)
        for c in range(n_main, n_chunks):
            b = c % NB
            pltpu.make_async_copy(
                glove_hbm.at[gidxs[b]],
                rows_v.at[b], gsems[b]).wait()
            patch(c, b)
            pltpu.async_copy(
                rows_v.at[b],
                out_hbm.at[pl.ds(w_base + c * S, S)], wsems[b])
        for b in range(NB):
            pltpu.make_async_copy(
                rows_v.at[b], out_hbm.at[pl.ds(w_base, S)], wsems[b]).wait()

    out = k(context.reshape(-1).astype(jnp.int32), glove_table, unk_table)
    return out.reshape(B, T, D)


# speedup vs baseline: 1.0173x; 1.0173x over previous
import functools

import jax
import jax.numpy as jnp
from jax import lax
from jax.experimental import pallas as pl
from jax.experimental.pallas import tpu as pltpu
from jax.experimental.pallas import tpu_sc as plsc

UNK_SIZE = 1000


def _lane_shuffle(x, perm_idx):
    return lax.gather(
        x, perm_idx[:, None],
        dimension_numbers=lax.GatherDimensionNumbers(
            offset_dims=(), collapsed_slice_dims=(0,), start_index_map=(0,)),
        slice_sizes=(1,),
        mode=lax.GatherScatterMode.PROMISE_IN_BOUNDS)


def kernel(context, glove_table, unk_table):
    B, T = context.shape
    V, D = glove_table.shape
    TOK = B * T

    info = plsc.get_sparse_core_info()
    NC, NS, L = info.num_cores, info.num_subcores, info.num_lanes
    NW = NC * NS
    per_w = TOK // NW
    S = 128
    n_chunks = per_w // S
    NB = 5
    assert TOK == per_w * NW and per_w == n_chunks * S and n_chunks % NB == 0

    mesh = plsc.VectorSubcoreMesh(core_axis_name="c", subcore_axis_name="s")

    @functools.partial(
        pl.kernel,
        mesh=mesh,
        out_type=jax.ShapeDtypeStruct((TOK, D), jnp.float32),
        scratch_types=[
            pltpu.VMEM((per_w,), jnp.int32),
            pltpu.VMEM((n_chunks, S), jnp.int32),
            pltpu.VMEM((NB, S, D), jnp.float32),
            pltpu.VMEM((n_chunks * L,), jnp.int32),
        ] + [pltpu.SemaphoreType.DMA] * (2 * NB),
    )
    def k(ctx_hbm, glove_hbm, unk_hbm, out_hbm,
          idx_v, gidx_v, rows_v, min_v, *sems):
        gsems, wsems = sems[:NB], sems[NB:]
        wid = lax.axis_index("s") * NC + lax.axis_index("c")
        w_base = wid * per_w
        iota = lax.iota(jnp.int32, L)

        pltpu.sync_copy(ctx_hbm.at[pl.ds(w_base, per_w)], idx_v)

        def pre_body(c, carry):
            acc = jnp.full((L,), jnp.int32(2**31 - 1), jnp.int32)
            for g in range(S // L):
                v = idx_v[pl.ds(c * S + g * L, L)]
                gidx_v[c, pl.ds(g * L, L)] = jnp.maximum(v - UNK_SIZE, 0)
                acc = jnp.minimum(acc, v)
            for sh in (1, 2, 4, 8):
                acc = jnp.minimum(acc, _lane_shuffle(acc, iota ^ sh))
            min_v[pl.ds(c * L, L)] = acc
            return carry

        def start_gather(c, buf):
            return pltpu.async_copy(
                glove_hbm.at[gidx_v.at[c]], rows_v.at[buf], gsems[buf])

        for j in range(NB - 1):
            pre_body(j, 0)
            start_gather(j, j)
        lax.fori_loop(NB - 1, n_chunks, pre_body, 0)

        def patch(c, buf):
            mn = min_v[pl.ds(c * L, L)]

            @pl.when(mn[0] < UNK_SIZE)
            def _():
                def patch_group(g, carry):
                    vv = idx_v[pl.ds(c * S + g * L, L)]
                    for i in range(L):
                        uid = vv[i]

                        @pl.when(uid < UNK_SIZE)
                        def _(i=i, uid=uid):
                            pltpu.sync_copy(
                                unk_hbm.at[pl.ds(uid, 1)],
                                rows_v.at[buf].at[pl.ds(g * L + i, 1)],
                            )
                    return carry

                lax.fori_loop(0, S // L, patch_group, 0)

        def step_body(s, carry):
            for b in range(NB):
                c = s * NB + b
                b2 = (b + NB - 1) % NB

                @pl.when(c + NB - 1 < n_chunks)
                def _(c=c, b2=b2):
                    @pl.when(c >= 1)
                    def _():
                        pltpu.make_async_copy(
                            rows_v.at[b2],
                            out_hbm.at[pl.ds(w_base, S)],
                            wsems[b2]).wait()

                    start_gather(c + NB - 1, b2)

                pltpu.make_async_copy(
                    glove_hbm.at[gidx_v.at[c]],
                    rows_v.at[b], gsems[b]).wait()
                patch(c, b)
                pltpu.async_copy(
                    rows_v.at[b],
                    out_hbm.at[pl.ds(w_base + c * S, S)], wsems[b])
            return carry

        lax.fori_loop(0, n_chunks // NB, step_body, 0)
        for b in range(NB):
            pltpu.make_async_copy(
                rows_v.at[b], out_hbm.at[pl.ds(w_base, S)], wsems[b]).wait()

    out = k(context.reshape(-1).astype(jnp.int32), glove_table, unk_table)
    return out.reshape(B, T, D)
